# baseline (device time: 67693 ns/iter reference)
import os

import jax
import jax.numpy as jnp
from jax import lax
from jax.experimental import pallas as pl
from jax.experimental.pallas import tpu as pltpu

_VARIANT = os.environ.get("KERNEL_VARIANT", "full")
_SCOPED = os.environ.get("KERNEL_SCOPED", "0") == "1"

N_DEV = 8
SQ = 1024
SKV = 1024
H_PER = 8
DH = 128
DM = 1024
BLK = 64
CH = SQ // N_DEV
SCALE = 0.08838834764831843


def kernel(x, Wq, K_ext, V_ext, Wo):
    xb = x[0]
    k2 = K_ext[0].reshape(SKV, H_PER * DH)
    v2 = V_ext[0].reshape(SKV, H_PER * DH)

    def body(x_ref, k_ref, v_ref, wq_hbm, wo_hbm, out_ref,
             wq_vmem, wo_vmem, partials, contrib, gather,
             w_sems, p1_send_sems, p1_recv_sem, p2_send_sems, p2_recv_sem):
        my_pos = lax.axis_index("i")

        wq_dma = pltpu.make_async_copy(
            wq_hbm.at[:, pl.ds(my_pos * (H_PER * DH), H_PER * DH)],
            wq_vmem, w_sems.at[0])
        wo_dma = pltpu.make_async_copy(
            wo_hbm.at[pl.ds(my_pos * (H_PER * DH), H_PER * DH), :],
            wo_vmem, w_sems.at[1])
        wq_dma.start()
        wo_dma.start()

        barrier_sem = pltpu.get_barrier_semaphore()
        for p in range(N_DEV):
            @pl.when(p != my_pos)
            def _():
                pl.semaphore_signal(
                    barrier_sem, inc=1,
                    device_id=(p,), device_id_type=pl.DeviceIdType.MESH,
                )
        pl.semaphore_wait(barrier_sem, N_DEV - 1)

        wq_dma.wait()
        wo_dma.wait()
        wq_bf = (wq_vmem[:, :] * SCALE).astype(jnp.bfloat16)
        wo_bf = wo_vmem[:, :].astype(jnp.bfloat16)

        def compute_piece(c):
            row0, kvc = c * CH, (c + 1) * CH
            xq = x_ref[row0:row0 + CH, :].astype(jnp.bfloat16)
            q = jnp.dot(
                xq, wq_bf, preferred_element_type=jnp.float32
            ).astype(jnp.bfloat16)
            qb = (row0 + lax.broadcasted_iota(jnp.int32, (CH, kvc), 0)) // BLK
            kb = lax.broadcasted_iota(jnp.int32, (CH, kvc), 1) // BLK
            mask = kb <= qb
            ctx_heads = []
            for h in range(H_PER):
                sl = slice(h * DH, (h + 1) * DH)
                k_h = k_ref[0:kvc, sl].astype(jnp.bfloat16)
                s = lax.dot_general(
                    q[:, sl], k_h, (((1,), (1,)), ((), ())),
                    preferred_element_type=jnp.float32,
                )
                e = jnp.where(mask, jnp.exp(s), 0.0)
                w = (e / jnp.sum(e, axis=1, keepdims=True)).astype(jnp.bfloat16)
                ctx_heads.append(jnp.dot(
                    w, v_ref[0:kvc, sl].astype(jnp.bfloat16),
                    preferred_element_type=jnp.float32,
                ).astype(jnp.bfloat16))
            ctx = jnp.concatenate(ctx_heads, axis=1)
            acc = jnp.dot(ctx, wo_bf, preferred_element_type=jnp.float32)
            partials[row0:row0 + CH, :] = acc.astype(jnp.bfloat16)

        def p1_send(c):
            @pl.when(c != my_pos)
            def _():
                rdma = pltpu.make_async_remote_copy(
                    src_ref=partials.at[pl.ds(c * CH, CH), :],
                    dst_ref=contrib.at[my_pos],
                    send_sem=p1_send_sems.at[c],
                    recv_sem=p1_recv_sem.at[0],
                    device_id=(c,),
                    device_id_type=pl.DeviceIdType.MESH,
                )
                rdma.start()

            @pl.when(c == my_pos)
            def _():
                contrib[c, :, :] = partials[c * CH:(c + 1) * CH, :]

        def p1_recv_wait():
            rdma = pltpu.make_async_remote_copy(
                src_ref=contrib.at[0],
                dst_ref=contrib.at[0],
                send_sem=p1_send_sems.at[0],
                recv_sem=p1_recv_sem.at[0],
                device_id=(0,),
                device_id_type=pl.DeviceIdType.MESH,
            )
            for _ in range(N_DEV - 1):
                rdma.wait_recv()

        def reduce_and_broadcast():
            p1_recv_wait()
            red = contrib[0, :, :].astype(jnp.float32)
            for s in range(1, N_DEV):
                red = red + contrib[s, :, :].astype(jnp.float32)
            gather[pl.ds(my_pos * CH, CH), :] = red.astype(jnp.bfloat16)
            for j in range(N_DEV):
                @pl.when(j != my_pos)
                def _():
                    rdma = pltpu.make_async_remote_copy(
                        src_ref=gather.at[pl.ds(my_pos * CH, CH), :],
                        dst_ref=gather.at[pl.ds(my_pos * CH, CH), :],
                        send_sem=p2_send_sems.at[j],
                        recv_sem=p2_recv_sem.at[0],
                        device_id=(j,),
                        device_id_type=pl.DeviceIdType.MESH,
                    )
                    rdma.start()

        def drain():
            for c in range(N_DEV):
                @pl.when(c != my_pos)
                def _():
                    pltpu.make_async_remote_copy(
                        src_ref=partials.at[pl.ds(c * CH, CH), :],
                        dst_ref=contrib.at[my_pos],
                        send_sem=p1_send_sems.at[c],
                        recv_sem=p1_recv_sem.at[0],
                        device_id=(c,),
                        device_id_type=pl.DeviceIdType.MESH,
                    ).wait_send()
            for j in range(N_DEV):
                @pl.when(j != my_pos)
                def _():
                    pltpu.make_async_remote_copy(
                        src_ref=gather.at[pl.ds(my_pos * CH, CH), :],
                        dst_ref=gather.at[pl.ds(my_pos * CH, CH), :],
                        send_sem=p2_send_sems.at[j],
                        recv_sem=p2_recv_sem.at[0],
                        device_id=(j,),
                        device_id_type=pl.DeviceIdType.MESH,
                    ).wait_send()

        if _VARIANT == "nocomm":
            for c in range(N_DEV):
                compute_piece(c)
            out_ref[0, :, :] = partials[:, :].astype(jnp.float32)
            return

        for c in range(6):
            compute_piece(c)
            p1_send(c)
        @pl.when(my_pos < 4)
        def _():
            reduce_and_broadcast()
        for c in range(6, N_DEV):
            compute_piece(c)
            p1_send(c)
        @pl.when(my_pos >= 4)
        def _():
            reduce_and_broadcast()

        tail = pltpu.make_async_remote_copy(
            src_ref=gather.at[pl.ds(my_pos * CH, CH), :],
            dst_ref=gather.at[pl.ds(my_pos * CH, CH), :],
            send_sem=p2_send_sems.at[0],
            recv_sem=p2_recv_sem.at[0],
            device_id=(0,),
            device_id_type=pl.DeviceIdType.MESH,
        )
        for _ in range(N_DEV - 1):
            tail.wait_recv()

        drain()
        out_ref[0, :, :] = gather[:, :].astype(jnp.float32)

    out = pl.pallas_call(
        body,
        out_shape=jax.ShapeDtypeStruct((1, SQ, DM), jnp.float32),
        in_specs=[
            pl.BlockSpec(memory_space=pltpu.VMEM),
            pl.BlockSpec(memory_space=pltpu.VMEM),
            pl.BlockSpec(memory_space=pltpu.VMEM),
            pl.BlockSpec(memory_space=pl.ANY),
            pl.BlockSpec(memory_space=pl.ANY),
        ],
        out_specs=pl.BlockSpec(memory_space=pltpu.VMEM),
        scratch_shapes=[
            pltpu.VMEM((DM, H_PER * DH), jnp.float32),
            pltpu.VMEM((H_PER * DH, DM), jnp.float32),
            pltpu.VMEM((SQ, DM), jnp.bfloat16),
            pltpu.VMEM((N_DEV, CH, DM), jnp.bfloat16),
            pltpu.VMEM((SQ, DM), jnp.bfloat16),
            pltpu.SemaphoreType.DMA((2,)),
            pltpu.SemaphoreType.DMA((N_DEV,)),
            pltpu.SemaphoreType.DMA((1,)),
            pltpu.SemaphoreType.DMA((N_DEV,)),
            pltpu.SemaphoreType.DMA((1,)),
        ],
        compiler_params=pltpu.CompilerParams(collective_id=0),
    )(xb, k2, v2, Wq, Wo)
    return out
